# baseline (speedup 1.0000x reference)
import functools

import jax
import jax.numpy as jnp
from jax import lax
from jax.experimental import pallas as pl
from jax.experimental.pallas import tpu as pltpu
from jax.experimental.pallas import tpu_sc as plsc

N_BINS = 15
N_ROWS = 1_000_000
N_CLS = 100

TC_BLOCK = 65536
TC_GRID = 16

LANES = 16
N_TILES = 16
UNROLL = 4
BANKS = UNROLL
SLOTS = 16 * LANES
PAD_N = TC_GRID * TC_BLOCK
PER_TILE = PAD_N // N_TILES
VREGS_PER_TILE = PER_TILE // LANES


def _tc_body(x_ref, lab_ref, out_ref):
    xt = x_ref[...]
    lab = lab_ref[...]
    m = jnp.max(xt, axis=0)
    e = jnp.exp(xt - m[None, :])
    s = jnp.sum(e, axis=0)
    row = lax.broadcasted_iota(jnp.int32, xt.shape, 0)
    hit = jnp.where((row == lab[None, :]) & (xt == m[None, :]), 1.0, 0.0)
    cnt = jnp.sum(hit, axis=0)
    conf = 1.0 / s
    packed = jnp.where(cnt > 0.5, -conf, conf)
    row_id = pl.program_id(0) * TC_BLOCK + lax.iota(jnp.int32, TC_BLOCK)
    out_ref[...] = jnp.where(row_id < N_ROWS, packed, 0.0)


def _tc_stage(logits, labels):
    return pl.pallas_call(
        _tc_body,
        grid=(TC_GRID,),
        in_specs=[
            pl.BlockSpec((N_CLS, TC_BLOCK), lambda i: (0, i)),
            pl.BlockSpec((TC_BLOCK,), lambda i: (i,)),
        ],
        out_specs=pl.BlockSpec((TC_BLOCK,), lambda i: (i,)),
        out_shape=jax.ShapeDtypeStruct((PAD_N,), jnp.float32),
    )(logits.T, labels.astype(jnp.int32))


def _sc_body(packed_hbm, out_hbm, buf_v, comb_v, sconf_v, cnt_v, acc_v, conf_v,
             outbuf_v, cnt_sh, sconf_sh, sacc_sh):
    tid = lax.axis_index("s")
    pltpu.sync_copy(packed_hbm.at[pl.ds(tid * PER_TILE, PER_TILE)], buf_v)

    lane = lax.iota(jnp.int32, LANES)
    zeros = jnp.zeros((LANES,), jnp.float32)
    for r in range(16 * BANKS):
        comb_v[pl.ds(r * LANES, LANES)] = zeros
        sconf_v[pl.ds(r * LANES, LANES)] = zeros
    bases = [lane * 16 + (1 + u * SLOTS) for u in range(UNROLL)]

    @plsc.parallel_loop(0, VREGS_PER_TILE, step=UNROLL, unroll=2)
    def _main(k):
        off = k * LANES
        for u in range(UNROLL):
            v = buf_v[pl.ds(off + u * LANES, LANES)]
            conf = jnp.abs(v)
            live = conf > 0.0
            t = conf * jnp.float32(N_BINS)
            b = jnp.minimum(t.astype(jnp.int32), N_BINS - 1)
            slot = bases[u] + b
            comb = jnp.where(v < 0.0, jnp.float32(4097.0), jnp.float32(1.0))
            plsc.addupdate_scatter(comb_v, [slot], comb, mask=live)
            plsc.addupdate_scatter(sconf_v, [slot], conf, mask=live)

    for r in range(16):
        c = comb_v[pl.ds(r * LANES, LANES)]
        f = sconf_v[pl.ds(r * LANES, LANES)]
        for u in range(1, BANKS):
            c = c + comb_v[pl.ds(u * SLOTS + r * LANES, LANES)]
            f = f + sconf_v[pl.ds(u * SLOTS + r * LANES, LANES)]
        a = (c * jnp.float32(1.0 / 4096.0)).astype(jnp.int32).astype(jnp.float32)
        cnt_v[pl.ds(r * LANES, LANES)] = c - a * 4096.0
        acc_v[pl.ds(r * LANES, LANES)] = a
        conf_v[pl.ds(r * LANES, LANES)] = f

    pltpu.sync_copy(cnt_v, cnt_sh.at[pl.ds(tid * SLOTS, SLOTS)])
    pltpu.sync_copy(conf_v, sconf_sh.at[pl.ds(tid * SLOTS, SLOTS)])
    pltpu.sync_copy(acc_v, sacc_sh.at[pl.ds(tid * SLOTS, SLOTS)])

    plsc.subcore_barrier()

    @pl.when(tid == 0)
    def _finalize():
        nslots = N_TILES * SLOTS
        pltpu.sync_copy(cnt_sh, buf_v.at[pl.ds(0, nslots)])
        pltpu.sync_copy(sconf_sh, buf_v.at[pl.ds(nslots, nslots)])
        pltpu.sync_copy(sacc_sh, buf_v.at[pl.ds(2 * nslots, nslots)])
        cnt = zeros
        sc = zeros
        sa = zeros
        for r in range(nslots // LANES):
            cnt = cnt + buf_v[pl.ds(r * LANES, LANES)]
            sc = sc + buf_v[pl.ds(nslots + r * LANES, LANES)]
            sa = sa + buf_v[pl.ds(2 * nslots + r * LANES, LANES)]
        safe = jnp.maximum(cnt, 1.0)
        contrib = jnp.abs(sc / safe - sa / safe) * (cnt * jnp.float32(1.0 / N_ROWS))
        valid = (cnt > 0.0) & (lane > 0)
        contrib = jnp.where(valid, contrib, 0.0)
        outbuf_v[...] = jnp.broadcast_to(jnp.sum(contrib), (LANES,))
        pltpu.sync_copy(outbuf_v, out_hbm)


@functools.cache
def _sc_histogram_fn():
    return functools.partial(
        pl.kernel,
        out_type=jax.ShapeDtypeStruct((LANES,), jnp.float32),
        mesh=plsc.VectorSubcoreMesh(
            core_axis_name="c", subcore_axis_name="s", num_cores=1),
        compiler_params=pltpu.CompilerParams(needs_layout_passes=False),
        scratch_types=[
            pltpu.VMEM((PER_TILE,), jnp.float32),
            pltpu.VMEM((BANKS * SLOTS,), jnp.float32),
            pltpu.VMEM((BANKS * SLOTS,), jnp.float32),
            pltpu.VMEM((SLOTS,), jnp.float32),
            pltpu.VMEM((SLOTS,), jnp.float32),
            pltpu.VMEM((SLOTS,), jnp.float32),
            pltpu.VMEM((LANES,), jnp.float32),
            pltpu.VMEM_SHARED((N_TILES * SLOTS,), jnp.float32),
            pltpu.VMEM_SHARED((N_TILES * SLOTS,), jnp.float32),
            pltpu.VMEM_SHARED((N_TILES * SLOTS,), jnp.float32),
        ],
    )(_sc_body)


@jax.jit
def kernel(logits, labels):
    padded = _tc_stage(logits, labels)
    ece_vec = _sc_histogram_fn()(padded)
    return ece_vec[0:1]


# speedup vs baseline: 1.0348x; 1.0348x over previous
import functools

import jax
import jax.numpy as jnp
from jax import lax
from jax.experimental import pallas as pl
from jax.experimental.pallas import tpu as pltpu
from jax.experimental.pallas import tpu_sc as plsc

N_BINS = 15
N_ROWS = 1_000_000
N_CLS = 100

TC_BLOCK = 16384
TC_GRID = 62

LANES = 16
N_TILES = 16
UNROLL = 4
BANKS = UNROLL
SLOTS = 16 * LANES
PAD_N = TC_GRID * TC_BLOCK
PER_TILE = PAD_N // N_TILES
VREGS_PER_TILE = PER_TILE // LANES


def _tc_body(x_ref, lab_ref, out_ref):
    xt = x_ref[...]
    lab = lab_ref[...]
    m = jnp.max(xt, axis=0)
    e = jnp.exp(xt - m[None, :])
    s = jnp.sum(e, axis=0)
    row = lax.broadcasted_iota(jnp.int32, xt.shape, 0)
    hit = jnp.where((row == lab[None, :]) & (xt == m[None, :]), 1.0, 0.0)
    cnt = jnp.sum(hit, axis=0)
    conf = 1.0 / s
    packed = jnp.where(cnt > 0.5, -conf, conf)
    row_id = pl.program_id(0) * TC_BLOCK + lax.iota(jnp.int32, TC_BLOCK)
    out_ref[...] = jnp.where(row_id < N_ROWS, packed, 0.0)


def _tc_stage(logits, labels):
    return pl.pallas_call(
        _tc_body,
        grid=(TC_GRID,),
        in_specs=[
            pl.BlockSpec((N_CLS, TC_BLOCK), lambda i: (0, i)),
            pl.BlockSpec((TC_BLOCK,), lambda i: (i,)),
        ],
        out_specs=pl.BlockSpec((TC_BLOCK,), lambda i: (i,)),
        out_shape=jax.ShapeDtypeStruct((PAD_N,), jnp.float32),
    )(logits.T, labels.astype(jnp.int32))


def _sc_body(packed_hbm, out_hbm, buf_v, comb_v, sconf_v, cnt_v, acc_v, conf_v,
             outbuf_v, cnt_sh, sconf_sh, sacc_sh):
    tid = lax.axis_index("s")
    pltpu.sync_copy(packed_hbm.at[pl.ds(tid * PER_TILE, PER_TILE)], buf_v)

    lane = lax.iota(jnp.int32, LANES)
    zeros = jnp.zeros((LANES,), jnp.float32)
    for r in range(16 * BANKS):
        comb_v[pl.ds(r * LANES, LANES)] = zeros
        sconf_v[pl.ds(r * LANES, LANES)] = zeros
    bases = [lane * 16 + (1 + u * SLOTS) for u in range(UNROLL)]

    @plsc.parallel_loop(0, VREGS_PER_TILE, step=UNROLL, unroll=2)
    def _main(k):
        off = k * LANES
        for u in range(UNROLL):
            v = buf_v[pl.ds(off + u * LANES, LANES)]
            conf = jnp.abs(v)
            live = conf > 0.0
            t = conf * jnp.float32(N_BINS)
            b = jnp.minimum(t.astype(jnp.int32), N_BINS - 1)
            slot = bases[u] + b
            comb = jnp.where(v < 0.0, jnp.float32(4097.0), jnp.float32(1.0))
            plsc.addupdate_scatter(comb_v, [slot], comb, mask=live)
            plsc.addupdate_scatter(sconf_v, [slot], conf, mask=live)

    for r in range(16):
        c = comb_v[pl.ds(r * LANES, LANES)]
        f = sconf_v[pl.ds(r * LANES, LANES)]
        for u in range(1, BANKS):
            c = c + comb_v[pl.ds(u * SLOTS + r * LANES, LANES)]
            f = f + sconf_v[pl.ds(u * SLOTS + r * LANES, LANES)]
        a = (c * jnp.float32(1.0 / 4096.0)).astype(jnp.int32).astype(jnp.float32)
        cnt_v[pl.ds(r * LANES, LANES)] = c - a * 4096.0
        acc_v[pl.ds(r * LANES, LANES)] = a
        conf_v[pl.ds(r * LANES, LANES)] = f

    pltpu.sync_copy(cnt_v, cnt_sh.at[pl.ds(tid * SLOTS, SLOTS)])
    pltpu.sync_copy(conf_v, sconf_sh.at[pl.ds(tid * SLOTS, SLOTS)])
    pltpu.sync_copy(acc_v, sacc_sh.at[pl.ds(tid * SLOTS, SLOTS)])

    plsc.subcore_barrier()

    @pl.when(tid == 0)
    def _finalize():
        nslots = N_TILES * SLOTS
        pltpu.sync_copy(cnt_sh, buf_v.at[pl.ds(0, nslots)])
        pltpu.sync_copy(sconf_sh, buf_v.at[pl.ds(nslots, nslots)])
        pltpu.sync_copy(sacc_sh, buf_v.at[pl.ds(2 * nslots, nslots)])
        cnt = zeros
        sc = zeros
        sa = zeros
        for r in range(nslots // LANES):
            cnt = cnt + buf_v[pl.ds(r * LANES, LANES)]
            sc = sc + buf_v[pl.ds(nslots + r * LANES, LANES)]
            sa = sa + buf_v[pl.ds(2 * nslots + r * LANES, LANES)]
        safe = jnp.maximum(cnt, 1.0)
        contrib = jnp.abs(sc / safe - sa / safe) * (cnt * jnp.float32(1.0 / N_ROWS))
        valid = (cnt > 0.0) & (lane > 0)
        contrib = jnp.where(valid, contrib, 0.0)
        outbuf_v[...] = jnp.broadcast_to(jnp.sum(contrib), (LANES,))
        pltpu.sync_copy(outbuf_v, out_hbm)


@functools.cache
def _sc_histogram_fn():
    return functools.partial(
        pl.kernel,
        out_type=jax.ShapeDtypeStruct((LANES,), jnp.float32),
        mesh=plsc.VectorSubcoreMesh(
            core_axis_name="c", subcore_axis_name="s", num_cores=1),
        compiler_params=pltpu.CompilerParams(needs_layout_passes=False),
        scratch_types=[
            pltpu.VMEM((PER_TILE,), jnp.float32),
            pltpu.VMEM((BANKS * SLOTS,), jnp.float32),
            pltpu.VMEM((BANKS * SLOTS,), jnp.float32),
            pltpu.VMEM((SLOTS,), jnp.float32),
            pltpu.VMEM((SLOTS,), jnp.float32),
            pltpu.VMEM((SLOTS,), jnp.float32),
            pltpu.VMEM((LANES,), jnp.float32),
            pltpu.VMEM_SHARED((N_TILES * SLOTS,), jnp.float32),
            pltpu.VMEM_SHARED((N_TILES * SLOTS,), jnp.float32),
            pltpu.VMEM_SHARED((N_TILES * SLOTS,), jnp.float32),
        ],
    )(_sc_body)


@jax.jit
def kernel(logits, labels):
    padded = _tc_stage(logits, labels)
    ece_vec = _sc_histogram_fn()(padded)
    return ece_vec[0:1]


# speedup vs baseline: 1.0482x; 1.0130x over previous
import functools

import jax
import jax.numpy as jnp
from jax import lax
from jax.experimental import pallas as pl
from jax.experimental.pallas import tpu as pltpu
from jax.experimental.pallas import tpu_sc as plsc

N_BINS = 15
N_ROWS = 1_000_000
N_CLS = 100

TC_BLOCK = 32768
TC_GRID = 31

LANES = 16
N_TILES = 16
UNROLL = 4
BANKS = UNROLL
SLOTS = 16 * LANES
PAD_N = TC_GRID * TC_BLOCK
PER_TILE = PAD_N // N_TILES
VREGS_PER_TILE = PER_TILE // LANES


def _tc_body(x_ref, lab_ref, out_ref):
    xt = x_ref[...]
    lab = lab_ref[...]
    m = jnp.max(xt, axis=0)
    e = jnp.exp(xt - m[None, :])
    s = jnp.sum(e, axis=0)
    row = lax.broadcasted_iota(jnp.int32, xt.shape, 0)
    hit = jnp.where((row == lab[None, :]) & (xt == m[None, :]), 1.0, 0.0)
    cnt = jnp.sum(hit, axis=0)
    conf = 1.0 / s
    packed = jnp.where(cnt > 0.5, -conf, conf)
    row_id = pl.program_id(0) * TC_BLOCK + lax.iota(jnp.int32, TC_BLOCK)
    out_ref[...] = jnp.where(row_id < N_ROWS, packed, 0.0)


def _tc_stage(logits, labels):
    return pl.pallas_call(
        _tc_body,
        grid=(TC_GRID,),
        in_specs=[
            pl.BlockSpec((N_CLS, TC_BLOCK), lambda i: (0, i)),
            pl.BlockSpec((TC_BLOCK,), lambda i: (i,)),
        ],
        out_specs=pl.BlockSpec((TC_BLOCK,), lambda i: (i,)),
        out_shape=jax.ShapeDtypeStruct((PAD_N,), jnp.float32),
    )(logits.T, labels.astype(jnp.int32))


def _sc_body(packed_hbm, out_hbm, buf_v, comb_v, sconf_v, cnt_v, acc_v, conf_v,
             outbuf_v, cnt_sh, sconf_sh, sacc_sh):
    tid = lax.axis_index("s")
    pltpu.sync_copy(packed_hbm.at[pl.ds(tid * PER_TILE, PER_TILE)], buf_v)

    lane = lax.iota(jnp.int32, LANES)
    zeros = jnp.zeros((LANES,), jnp.float32)
    for r in range(16 * BANKS):
        comb_v[pl.ds(r * LANES, LANES)] = zeros
        sconf_v[pl.ds(r * LANES, LANES)] = zeros
    bases = [lane * 16 + (1 + u * SLOTS) for u in range(UNROLL)]

    @plsc.parallel_loop(0, VREGS_PER_TILE, step=UNROLL, unroll=2)
    def _main(k):
        off = k * LANES
        for u in range(UNROLL):
            v = buf_v[pl.ds(off + u * LANES, LANES)]
            conf = jnp.abs(v)
            live = conf > 0.0
            t = conf * jnp.float32(N_BINS)
            b = jnp.minimum(t.astype(jnp.int32), N_BINS - 1)
            slot = bases[u] + b
            comb = jnp.where(v < 0.0, jnp.float32(4097.0), jnp.float32(1.0))
            plsc.addupdate_scatter(comb_v, [slot], comb, mask=live)
            plsc.addupdate_scatter(sconf_v, [slot], conf, mask=live)

    for r in range(16):
        c = comb_v[pl.ds(r * LANES, LANES)]
        f = sconf_v[pl.ds(r * LANES, LANES)]
        for u in range(1, BANKS):
            c = c + comb_v[pl.ds(u * SLOTS + r * LANES, LANES)]
            f = f + sconf_v[pl.ds(u * SLOTS + r * LANES, LANES)]
        a = (c * jnp.float32(1.0 / 4096.0)).astype(jnp.int32).astype(jnp.float32)
        cnt_v[pl.ds(r * LANES, LANES)] = c - a * 4096.0
        acc_v[pl.ds(r * LANES, LANES)] = a
        conf_v[pl.ds(r * LANES, LANES)] = f

    pltpu.sync_copy(cnt_v, cnt_sh.at[pl.ds(tid * SLOTS, SLOTS)])
    pltpu.sync_copy(conf_v, sconf_sh.at[pl.ds(tid * SLOTS, SLOTS)])
    pltpu.sync_copy(acc_v, sacc_sh.at[pl.ds(tid * SLOTS, SLOTS)])

    plsc.subcore_barrier()

    @pl.when(tid == 0)
    def _finalize():
        nslots = N_TILES * SLOTS
        pltpu.sync_copy(cnt_sh, buf_v.at[pl.ds(0, nslots)])
        pltpu.sync_copy(sconf_sh, buf_v.at[pl.ds(nslots, nslots)])
        pltpu.sync_copy(sacc_sh, buf_v.at[pl.ds(2 * nslots, nslots)])
        cnt = zeros
        sc = zeros
        sa = zeros
        for r in range(nslots // LANES):
            cnt = cnt + buf_v[pl.ds(r * LANES, LANES)]
            sc = sc + buf_v[pl.ds(nslots + r * LANES, LANES)]
            sa = sa + buf_v[pl.ds(2 * nslots + r * LANES, LANES)]
        safe = jnp.maximum(cnt, 1.0)
        contrib = jnp.abs(sc / safe - sa / safe) * (cnt * jnp.float32(1.0 / N_ROWS))
        valid = (cnt > 0.0) & (lane > 0)
        contrib = jnp.where(valid, contrib, 0.0)
        outbuf_v[...] = jnp.broadcast_to(jnp.sum(contrib), (LANES,))
        pltpu.sync_copy(outbuf_v, out_hbm)


@functools.cache
def _sc_histogram_fn():
    return functools.partial(
        pl.kernel,
        out_type=jax.ShapeDtypeStruct((LANES,), jnp.float32),
        mesh=plsc.VectorSubcoreMesh(
            core_axis_name="c", subcore_axis_name="s", num_cores=1),
        compiler_params=pltpu.CompilerParams(needs_layout_passes=False),
        scratch_types=[
            pltpu.VMEM((PER_TILE,), jnp.float32),
            pltpu.VMEM((BANKS * SLOTS,), jnp.float32),
            pltpu.VMEM((BANKS * SLOTS,), jnp.float32),
            pltpu.VMEM((SLOTS,), jnp.float32),
            pltpu.VMEM((SLOTS,), jnp.float32),
            pltpu.VMEM((SLOTS,), jnp.float32),
            pltpu.VMEM((LANES,), jnp.float32),
            pltpu.VMEM_SHARED((N_TILES * SLOTS,), jnp.float32),
            pltpu.VMEM_SHARED((N_TILES * SLOTS,), jnp.float32),
            pltpu.VMEM_SHARED((N_TILES * SLOTS,), jnp.float32),
        ],
    )(_sc_body)


@jax.jit
def kernel(logits, labels):
    padded = _tc_stage(logits, labels)
    ece_vec = _sc_histogram_fn()(padded)
    return ece_vec[0:1]
